# baseline (speedup 1.0000x reference)
import jax
import jax.numpy as jnp
from jax import lax
from jax.experimental import pallas as pl
from jax.experimental.pallas import tpu as pltpu
from jax.experimental.pallas import tpu_sc as plsc

_B = 4
_S = 4096
_C = 2048
_W = 8
_NCHUNK = 64
_RPW = _W * _NCHUNK
_WPB = 8
_NIN = 4
_HROW = 7


def _compute(buf, hbuf, obuf):
    @plsc.parallel_loop(0, _C // 16, unroll=4)
    def _(gi):
        o = gi * 16
        prev = hbuf[_HROW, pl.ds(o, 16)]
        for r in range(_W):
            cur = buf[r, pl.ds(o, 16)]
            obuf[r, pl.ds(o, 16)] = cur - prev
            prev = cur
        hbuf[_HROW, pl.ds(o, 16)] = prev


def _sc_body(x_hbm, out_hbm, buf0, buf1, buf2, buf3, obuf0, obuf1, hbuf,
             sem0, sem1, sem2, sem3, semo0, semo1, semh):
    cid = lax.axis_index("c")
    sid = lax.axis_index("s")
    wid = sid * 2 + cid
    b = wid // _WPB
    r0 = (wid % _WPB) * _RPW
    g0 = b * _S + r0

    bufs = (buf0, buf1, buf2, buf3)
    sems = (sem0, sem1, sem2, sem3)
    obufs = (obuf0, obuf1)
    osems = (semo0, semo1)

    def start_in(c, buf, sem):
        g = pl.multiple_of(g0 + c * _W, 8)
        pltpu.make_async_copy(x_hbm.at[pl.ds(g, _W)], buf, sem).start()

    def wait_in(buf, sem):
        pltpu.make_async_copy(x_hbm.at[pl.ds(0, _W)], buf, sem).wait()

    def start_out(c, obuf, osem):
        g = pl.multiple_of(g0 + c * _W, 8)
        pltpu.make_async_copy(obuf, out_hbm.at[pl.ds(g, _W)], osem).start()

    def wait_out(obuf, osem):
        pltpu.make_async_copy(obuf, out_hbm.at[pl.ds(g0, _W)], osem).wait()

    halo_hi = b * _S + (r0 + _S - 1) % _S + 1
    halo_lo = pl.multiple_of(halo_hi - 8, 8)
    pltpu.make_async_copy(x_hbm.at[pl.ds(halo_lo, 8)], hbuf, semh).start()
    for j in range(_NIN):
        start_in(j, bufs[j], sems[j])
    pltpu.make_async_copy(x_hbm.at[pl.ds(0, 8)], hbuf, semh).wait()

    def step(c, j, is_first):
        wait_in(bufs[j], sems[j])
        if not is_first:
            wait_out(obufs[j % 2], osems[j % 2])
        _compute(bufs[j], hbuf, obufs[j % 2])
        start_out(c, obufs[j % 2], osems[j % 2])

        @pl.when(c + _NIN < _NCHUNK)
        def _():
            start_in(c + _NIN, bufs[j], sems[j])

    def quad(k, carry):
        c0 = _NIN * k
        for j in range(_NIN):
            step(c0 + j, j, False)
        return carry

    for j in range(_NIN):
        step(j, j, j < 2)
    lax.fori_loop(1, _NCHUNK // _NIN, quad, 0)
    wait_out(obufs[0], osems[0])
    wait_out(obufs[1], osems[1])


def kernel(x):
    B, S, C = x.shape
    x2 = x.reshape(B * S, C)
    mesh = plsc.VectorSubcoreMesh(core_axis_name="c", subcore_axis_name="s")
    out = pl.kernel(
        _sc_body,
        out_type=jax.ShapeDtypeStruct((B * S, C), x.dtype),
        scratch_types=[
            pltpu.VMEM((_W, _C), jnp.float32),
            pltpu.VMEM((_W, _C), jnp.float32),
            pltpu.VMEM((_W, _C), jnp.float32),
            pltpu.VMEM((_W, _C), jnp.float32),
            pltpu.VMEM((_W, _C), jnp.float32),
            pltpu.VMEM((_W, _C), jnp.float32),
            pltpu.VMEM((8, _C), jnp.float32),
            pltpu.SemaphoreType.DMA,
            pltpu.SemaphoreType.DMA,
            pltpu.SemaphoreType.DMA,
            pltpu.SemaphoreType.DMA,
            pltpu.SemaphoreType.DMA,
            pltpu.SemaphoreType.DMA,
            pltpu.SemaphoreType.DMA,
        ],
        mesh=mesh,
    )(x2)
    return out.reshape(B, S, C)


# speedup vs baseline: 1.0237x; 1.0237x over previous
import jax
import jax.numpy as jnp
from jax import lax
from jax.experimental import pallas as pl
from jax.experimental.pallas import tpu as pltpu
from jax.experimental.pallas import tpu_sc as plsc

_B = 4
_S = 4096
_C = 2048
_W = 8
_NCHUNK = 64
_RPW = _W * _NCHUNK
_WPB = 8
_NIN = 4
_HROW = 7


def _compute(buf, hbuf, obuf):
    @plsc.parallel_loop(0, _C // 16)
    def _(gi):
        o = gi * 16
        prev = hbuf[_HROW, pl.ds(o, 16)]
        for r in range(_W):
            cur = buf[r, pl.ds(o, 16)]
            obuf[r, pl.ds(o, 16)] = cur - prev
            prev = cur
        hbuf[_HROW, pl.ds(o, 16)] = prev


def _sc_body(x_hbm, out_hbm, buf0, buf1, buf2, buf3, obuf0, obuf1, hbuf,
             sem0, sem1, sem2, sem3, semo0, semo1, semh):
    cid = lax.axis_index("c")
    sid = lax.axis_index("s")
    wid = sid * 2 + cid
    b = wid // _WPB
    r0 = (wid % _WPB) * _RPW
    g0 = b * _S + r0

    bufs = (buf0, buf1, buf2, buf3)
    sems = (sem0, sem1, sem2, sem3)
    obufs = (obuf0, obuf1)
    osems = (semo0, semo1)

    def start_in(c, buf, sem):
        g = pl.multiple_of(g0 + c * _W, 8)
        pltpu.make_async_copy(x_hbm.at[pl.ds(g, _W)], buf, sem).start()

    def wait_in(buf, sem):
        pltpu.make_async_copy(x_hbm.at[pl.ds(0, _W)], buf, sem).wait()

    def start_out(c, obuf, osem):
        g = pl.multiple_of(g0 + c * _W, 8)
        pltpu.make_async_copy(obuf, out_hbm.at[pl.ds(g, _W)], osem).start()

    def wait_out(obuf, osem):
        pltpu.make_async_copy(obuf, out_hbm.at[pl.ds(g0, _W)], osem).wait()

    halo_hi = b * _S + (r0 + _S - 1) % _S + 1
    halo_lo = pl.multiple_of(halo_hi - 8, 8)
    pltpu.make_async_copy(x_hbm.at[pl.ds(halo_lo, 8)], hbuf, semh).start()
    for j in range(_NIN):
        start_in(j, bufs[j], sems[j])
    pltpu.make_async_copy(x_hbm.at[pl.ds(0, 8)], hbuf, semh).wait()

    def step(c, j, is_first):
        wait_in(bufs[j], sems[j])
        if not is_first:
            wait_out(obufs[j % 2], osems[j % 2])
        _compute(bufs[j], hbuf, obufs[j % 2])
        start_out(c, obufs[j % 2], osems[j % 2])

        @pl.when(c + _NIN < _NCHUNK)
        def _():
            start_in(c + _NIN, bufs[j], sems[j])

    def quad(k, carry):
        c0 = _NIN * k
        for j in range(_NIN):
            step(c0 + j, j, False)
        return carry

    for j in range(_NIN):
        step(j, j, j < 2)
    lax.fori_loop(1, _NCHUNK // _NIN, quad, 0)
    wait_out(obufs[0], osems[0])
    wait_out(obufs[1], osems[1])


def kernel(x):
    B, S, C = x.shape
    x2 = x.reshape(B * S, C)
    mesh = plsc.VectorSubcoreMesh(core_axis_name="c", subcore_axis_name="s")
    out = pl.kernel(
        _sc_body,
        out_type=jax.ShapeDtypeStruct((B * S, C), x.dtype),
        scratch_types=[
            pltpu.VMEM((_W, _C), jnp.float32),
            pltpu.VMEM((_W, _C), jnp.float32),
            pltpu.VMEM((_W, _C), jnp.float32),
            pltpu.VMEM((_W, _C), jnp.float32),
            pltpu.VMEM((_W, _C), jnp.float32),
            pltpu.VMEM((_W, _C), jnp.float32),
            pltpu.VMEM((8, _C), jnp.float32),
            pltpu.SemaphoreType.DMA,
            pltpu.SemaphoreType.DMA,
            pltpu.SemaphoreType.DMA,
            pltpu.SemaphoreType.DMA,
            pltpu.SemaphoreType.DMA,
            pltpu.SemaphoreType.DMA,
            pltpu.SemaphoreType.DMA,
        ],
        mesh=mesh,
    )(x2)
    return out.reshape(B, S, C)
